# baseline (speedup 1.0000x reference)
import functools

import jax
import jax.numpy as jnp
from jax import lax
from jax.experimental import pallas as pl
from jax.experimental.pallas import tpu as pltpu
from jax.experimental.pallas import tpu_sc as plsc

L = 16


def kernel(users, jokes, mu_U, logvar_U, mu_V, logvar_V, z_U, z_V):
    B = users.shape[0]
    K = mu_U.shape[1]
    info = plsc.get_sparse_core_info()
    NC, NS = info.num_cores, info.num_subcores
    NW = NC * NS
    BPW = B // NW

    mesh = plsc.VectorSubcoreMesh(core_axis_name="c", subcore_axis_name="s")

    @functools.partial(
        pl.kernel,
        mesh=mesh,
        compiler_params=pltpu.CompilerParams(
            needs_layout_passes=False, use_tc_tiling_on_sc=False),
        out_type=jax.ShapeDtypeStruct((B,), jnp.float32),
        scratch_types=[
            pltpu.VMEM((BPW,), jnp.int32),
            pltpu.VMEM((BPW,), jnp.int32),
            pltpu.VMEM((BPW, K), jnp.float32),
            pltpu.VMEM((BPW, K), jnp.float32),
            pltpu.VMEM((BPW, K), jnp.float32),
            pltpu.VMEM((BPW, K), jnp.float32),
            pltpu.VMEM((BPW, K), jnp.float32),
            pltpu.VMEM((BPW, K), jnp.float32),
            pltpu.VMEM((BPW,), jnp.float32),
            pltpu.SemaphoreType.DMA,
        ],
    )
    def run(users_h, jokes_h, mu_u_h, lv_u_h, mu_v_h, lv_v_h, zu_h, zv_h,
            out_h, idx_u, idx_v, t_mu_u, t_lv_u, t_mu_v, t_lv_v, b_zu, b_zv,
            outv, sem):
        wid = lax.axis_index("s") * NC + lax.axis_index("c")
        base = wid * BPW

        pltpu.sync_copy(users_h.at[pl.ds(base, BPW)], idx_u)
        pltpu.sync_copy(jokes_h.at[pl.ds(base, BPW)], idx_v)
        cp1 = pltpu.async_copy(mu_u_h.at[idx_u], t_mu_u, sem)
        cp2 = pltpu.async_copy(lv_u_h.at[idx_u], t_lv_u, sem)
        cp3 = pltpu.async_copy(mu_v_h.at[idx_v], t_mu_v, sem)
        cp4 = pltpu.async_copy(lv_v_h.at[idx_v], t_lv_v, sem)
        pltpu.sync_copy(zu_h.at[pl.ds(base, BPW)], b_zu)
        pltpu.sync_copy(zv_h.at[pl.ds(base, BPW)], b_zv)
        cp1.wait()
        cp2.wait()
        cp3.wait()
        cp4.wait()

        lane = lax.iota(jnp.int32, L)

        def group(g, carry):
            acc = jnp.zeros((L,), jnp.float32)
            for j in range(L):
                b = g * L + j
                w = jnp.zeros((L,), jnp.float32)
                for h in range(K // L):
                    sl = pl.ds(h * L, L)
                    u = (b_zu[b, sl] * jnp.exp(t_lv_u[b, sl] * 0.5)
                         + t_mu_u[b, sl])
                    v = (b_zv[b, sl] * jnp.exp(t_lv_v[b, sl] * 0.5)
                         + t_mu_v[b, sl])
                    w = w + u * v
                acc = jnp.where(lane == j, jnp.sum(w), acc)
            outv[pl.ds(g * L, L)] = acc
            return carry

        lax.fori_loop(0, BPW // L, group, 0)
        pltpu.sync_copy(outv, out_h.at[pl.ds(base, BPW)])

    return run(users, jokes, mu_U, logvar_U, mu_V, logvar_V, z_U, z_V)
